# Initial kernel scaffold; baseline (speedup 1.0000x reference)
import jax
import jax.numpy as jnp
from jax.experimental import pallas as pl


def kernel(features, coords, Wq, Wk, Wv, Wo, Wpos, bpos, gamma, beta):
    raise NotImplementedError("write your pallas kernel here")


# speedup vs baseline: 1.0000x; 1.0000x over previous
import jax
import jax.numpy as jnp
from jax.experimental import pallas as pl

N = 10000
C = 256
NHEAD = 8
DIM = C // NHEAD
KNN_K = 16
EXTRA_K = 4
SCALE = DIM ** (-0.5)


def _knn(coords, k, chunk=1000):
    n = coords.shape[0]
    sq = jnp.sum(coords * coords, axis=1)
    idx_chunks = []
    for start in range(0, n, chunk):
        q = coords[start:start + chunk]
        d = jnp.sum(q * q, axis=1)[:, None] - 2.0 * (q @ coords.T) + sq[None, :]
        _, idx = jax.lax.top_k(-d, k)
        idx_chunks.append(idx)
    neighbor_idx = jnp.concatenate(idx_chunks, axis=0).reshape(-1)
    query_idx = jnp.repeat(jnp.arange(n, dtype=jnp.int32), k)
    return neighbor_idx.astype(jnp.int32), query_idx


def _edges(coords):
    n = coords.shape[0]
    neighbor_idx, query_idx = _knn(coords, KNN_K)
    ek = jax.random.key(12345)
    extra_q = jax.random.randint(jax.random.fold_in(ek, 0), (n * EXTRA_K,), 0, n, dtype=jnp.int32)
    extra_n = jax.random.randint(jax.random.fold_in(ek, 1), (n * EXTRA_K,), 0, n, dtype=jnp.int32)
    neighbor_idx = jnp.concatenate([neighbor_idx, extra_n], axis=0)
    query_idx = jnp.concatenate([query_idx, extra_q], axis=0)
    return neighbor_idx, query_idx


def _scatter_softmax(scores, seg, num):
    m = jax.ops.segment_max(scores, seg, num_segments=num)
    ex = jnp.exp(scores - m[seg])
    den = jax.ops.segment_sum(ex, seg, num_segments=num)
    return ex / den[seg]


def _ln_kernel(resid_ref, gamma_ref, beta_ref, out_ref):
    x = resid_ref[...]
    mu = jnp.mean(x, axis=-1, keepdims=True)
    var = jnp.mean((x - mu) ** 2, axis=-1, keepdims=True)
    normed = (x - mu) / jnp.sqrt(var + 1e-5)
    out_ref[...] = normed * gamma_ref[...][None, :] + beta_ref[...][None, :]


def kernel(features, coords, Wq, Wk, Wv, Wo, Wpos, bpos, gamma, beta):
    n = features.shape[0]
    neighbor_idx, query_idx = _edges(coords)
    Q = (features @ Wq.T).reshape(n, NHEAD, DIM)
    Kt = (features @ Wk.T).reshape(n, NHEAD, DIM)
    V = (features @ Wv.T).reshape(n, NHEAD, DIM)
    rel_pos = coords[query_idx] - coords[neighbor_idx]
    pos_enc = rel_pos @ Wpos.T + bpos[None, :]
    q = Q[query_idx]
    k = Kt[neighbor_idx]
    v = V[neighbor_idx]
    attn_scores = jnp.einsum('mhd,mhd->mh', q, k) * SCALE + pos_enc
    attn_weights = _scatter_softmax(attn_scores, query_idx, n)
    weighted = v * attn_weights[:, :, None]
    out = jax.ops.segment_sum(weighted, query_idx, num_segments=n)
    out = out.reshape(n, C)
    out_features = out @ Wo.T
    resid = out_features + features
    out = pl.pallas_call(
        _ln_kernel,
        grid=(10,),
        in_specs=[
            pl.BlockSpec((1000, C), lambda i: (i, 0)),
            pl.BlockSpec((C,), lambda i: (0,)),
            pl.BlockSpec((C,), lambda i: (0,)),
        ],
        out_specs=pl.BlockSpec((1000, C), lambda i: (i, 0)),
        out_shape=jax.ShapeDtypeStruct((n, C), jnp.float32),
    )(resid, gamma, beta)
    return out


# speedup vs baseline: 1.2452x; 1.2451x over previous
import jax
import jax.numpy as jnp
from jax.experimental import pallas as pl

N = 10000
C = 256
NHEAD = 8
DIM = C // NHEAD
KNN_K = 16
EXTRA_K = 4
SCALE = DIM ** (-0.5)


QB = 200


def _knn_block_kernel(cq_ref, cT_ref, sqq_ref, sqc_ref, out_ref):
    n = cT_ref.shape[1]
    qc = jnp.dot(cq_ref[...].astype(jnp.bfloat16), cT_ref[...].astype(jnp.bfloat16),
                 preferred_element_type=jnp.float32)
    d = sqq_ref[...] - 2.0 * qc + sqc_ref[...]
    iota = jax.lax.broadcasted_iota(jnp.int32, (1, n), 1)
    cols = []
    for _ in range(KNN_K):
        m = jnp.min(d, axis=1, keepdims=True)
        idx = jnp.min(jnp.where(d == m, iota, n), axis=1, keepdims=True)
        cols.append(idx)
        d = jnp.where(iota == idx, jnp.inf, d)
    pad = jnp.zeros((cq_ref.shape[0], 128 - KNN_K), jnp.int32)
    out_ref[...] = jnp.concatenate(cols + [pad], axis=1)


def _knn(coords, k, chunk=1000):
    n = coords.shape[0]
    coordsT = coords.T
    sq = jnp.sum(coords * coords, axis=1)
    idx_pad = pl.pallas_call(
        _knn_block_kernel,
        grid=(n // QB,),
        in_specs=[
            pl.BlockSpec((QB, 3), lambda i: (i, 0)),
            pl.BlockSpec((3, n), lambda i: (0, 0)),
            pl.BlockSpec((QB, 1), lambda i: (i, 0)),
            pl.BlockSpec((1, n), lambda i: (0, 0)),
        ],
        out_specs=pl.BlockSpec((QB, 128), lambda i: (i, 0)),
        out_shape=jax.ShapeDtypeStruct((n, 128), jnp.int32),
    )(coords, coordsT, sq.reshape(n, 1), sq.reshape(1, n))
    idx = idx_pad[:, :KNN_K]
    neighbor_idx = idx.reshape(-1)
    query_idx = jnp.repeat(jnp.arange(n, dtype=jnp.int32), k)
    return neighbor_idx.astype(jnp.int32), query_idx


def _edges(coords):
    n = coords.shape[0]
    neighbor_idx, query_idx = _knn(coords, KNN_K)
    ek = jax.random.key(12345)
    extra_q = jax.random.randint(jax.random.fold_in(ek, 0), (n * EXTRA_K,), 0, n, dtype=jnp.int32)
    extra_n = jax.random.randint(jax.random.fold_in(ek, 1), (n * EXTRA_K,), 0, n, dtype=jnp.int32)
    neighbor_idx = jnp.concatenate([neighbor_idx, extra_n], axis=0)
    query_idx = jnp.concatenate([query_idx, extra_q], axis=0)
    return neighbor_idx, query_idx


def _scatter_softmax(scores, seg, num):
    m = jax.ops.segment_max(scores, seg, num_segments=num)
    ex = jnp.exp(scores - m[seg])
    den = jax.ops.segment_sum(ex, seg, num_segments=num)
    return ex / den[seg]


def _ln_kernel(resid_ref, gamma_ref, beta_ref, out_ref):
    x = resid_ref[...]
    mu = jnp.mean(x, axis=-1, keepdims=True)
    var = jnp.mean((x - mu) ** 2, axis=-1, keepdims=True)
    normed = (x - mu) / jnp.sqrt(var + 1e-5)
    out_ref[...] = normed * gamma_ref[...][None, :] + beta_ref[...][None, :]


def kernel(features, coords, Wq, Wk, Wv, Wo, Wpos, bpos, gamma, beta):
    n = features.shape[0]
    neighbor_idx, query_idx = _edges(coords)
    Q = (features @ Wq.T).reshape(n, NHEAD, DIM)
    Kt = (features @ Wk.T).reshape(n, NHEAD, DIM)
    V = (features @ Wv.T).reshape(n, NHEAD, DIM)
    rel_pos = coords[query_idx] - coords[neighbor_idx]
    pos_enc = rel_pos @ Wpos.T + bpos[None, :]
    q = Q[query_idx]
    k = Kt[neighbor_idx]
    v = V[neighbor_idx]
    attn_scores = jnp.einsum('mhd,mhd->mh', q, k) * SCALE + pos_enc
    attn_weights = _scatter_softmax(attn_scores, query_idx, n)
    weighted = v * attn_weights[:, :, None]
    out = jax.ops.segment_sum(weighted, query_idx, num_segments=n)
    out = out.reshape(n, C)
    out_features = out @ Wo.T
    resid = out_features + features
    out = pl.pallas_call(
        _ln_kernel,
        grid=(10,),
        in_specs=[
            pl.BlockSpec((1000, C), lambda i: (i, 0)),
            pl.BlockSpec((C,), lambda i: (0,)),
            pl.BlockSpec((C,), lambda i: (0,)),
        ],
        out_specs=pl.BlockSpec((1000, C), lambda i: (i, 0)),
        out_shape=jax.ShapeDtypeStruct((n, C), jnp.float32),
    )(resid, gamma, beta)
    return out


# speedup vs baseline: 4.6147x; 3.7061x over previous
import functools

import jax
import jax.numpy as jnp
import numpy as np
from jax import lax
from jax.experimental import pallas as pl
from jax.experimental.pallas import tpu as pltpu
from jax.experimental.pallas import tpu_sc as plsc

N = 10000
NPAD = 10240
C = 256
NHEAD = 8
DIM = C // NHEAD
KNN_K = 16
EXTRA_K = 4
SCALE = DIM ** (-0.5)
ESLOT = 32
QB = 256
NBLK = NPAD // QB
QBK = 128
NEG = -1e30


def _proj_kernel(f_ref, w_ref, out_ref):
    out_ref[...] = jnp.dot(f_ref[...].astype(jnp.bfloat16),
                           w_ref[...].astype(jnp.bfloat16),
                           preferred_element_type=jnp.float32)


def _project(feat_pad, wqkv_t):
    return pl.pallas_call(
        _proj_kernel,
        grid=(NPAD // 512,),
        in_specs=[
            pl.BlockSpec((512, C), lambda i: (i, 0)),
            pl.BlockSpec((C, 3 * C), lambda i: (0, 0)),
        ],
        out_specs=pl.BlockSpec((512, 3 * C), lambda i: (i, 0)),
        out_shape=jax.ShapeDtypeStruct((NPAD, 3 * C), jnp.float32),
    )(feat_pad, wqkv_t)


def _knn_block_kernel(cq_ref, cT_ref, sqq_ref, sqc_ref, out_ref):
    n = cT_ref.shape[1]
    qc = jnp.dot(cq_ref[...].astype(jnp.bfloat16), cT_ref[...].astype(jnp.bfloat16),
                 preferred_element_type=jnp.float32)
    d = sqq_ref[...] - 2.0 * qc + sqc_ref[...]
    iota = jax.lax.broadcasted_iota(jnp.int32, (1, n), 1)
    cols = []
    for _ in range(KNN_K):
        m = jnp.min(d, axis=1, keepdims=True)
        idx = jnp.min(jnp.where(d == m, iota, n), axis=1, keepdims=True)
        cols.append(idx)
        d = jnp.where(iota == idx, jnp.inf, d)
    pad = jnp.zeros((cq_ref.shape[0], 128 - KNN_K), jnp.int32)
    out_ref[...] = jnp.concatenate(cols + [pad], axis=1)


def _knn_idx(coords_pad, coordsT, sq_col, sq_row):
    idx_pad = pl.pallas_call(
        _knn_block_kernel,
        grid=(NPAD // QBK,),
        in_specs=[
            pl.BlockSpec((QBK, 3), lambda i: (i, 0)),
            pl.BlockSpec((3, N), lambda i: (0, 0)),
            pl.BlockSpec((QBK, 1), lambda i: (i, 0)),
            pl.BlockSpec((1, N), lambda i: (0, 0)),
        ],
        out_specs=pl.BlockSpec((QBK, 128), lambda i: (i, 0)),
        out_shape=jax.ShapeDtypeStruct((NPAD, 128), jnp.int32),
    )(coords_pad[:, :3], coordsT, sq_col, sq_row)
    return idx_pad[:, :KNN_K]


NW = 32
EROWS = ESLOT * NPAD
PER_W = EROWS // NW
CH = 64


def _sc_gather(kmat, vmat, cmat, eidx):
    mesh = plsc.VectorSubcoreMesh(core_axis_name="c", subcore_axis_name="s")

    @functools.partial(
        pl.kernel, mesh=mesh,
        out_type=[
            jax.ShapeDtypeStruct((EROWS, C), jnp.float32),
            jax.ShapeDtypeStruct((EROWS, C), jnp.float32),
            jax.ShapeDtypeStruct((EROWS, 128), jnp.float32),
        ],
        scratch_types=[
            pltpu.VMEM((CH,), jnp.int32),
            pltpu.VMEM((CH, C), jnp.float32),
            pltpu.VMEM((CH, C), jnp.float32),
            pltpu.VMEM((CH, 128), jnp.float32),
            pltpu.SemaphoreType.DMA,
            pltpu.SemaphoreType.DMA,
            pltpu.SemaphoreType.DMA,
        ],
    )
    def gather_k(k_hbm, v_hbm, c_hbm, e_hbm, kg_hbm, vg_hbm, cg_hbm,
                 idx_v, kbuf, vbuf, cbuf, sk, sv, sc):
        wid = lax.axis_index("s") * 2 + lax.axis_index("c")
        base = wid * PER_W

        def body(j, carry):
            off = base + j * CH
            pltpu.sync_copy(e_hbm.at[pl.ds(off, CH)], idx_v)
            a = pltpu.async_copy(k_hbm.at[idx_v], kbuf, sk)
            b = pltpu.async_copy(v_hbm.at[idx_v], vbuf, sv)
            c = pltpu.async_copy(c_hbm.at[idx_v], cbuf, sc)
            a.wait()
            b.wait()
            c.wait()
            pltpu.sync_copy(kbuf, kg_hbm.at[pl.ds(off, CH)])
            pltpu.sync_copy(vbuf, vg_hbm.at[pl.ds(off, CH)])
            pltpu.sync_copy(cbuf, cg_hbm.at[pl.ds(off, CH)])
            return carry

        lax.fori_loop(0, PER_W // CH, body, 0)

    return gather_k(kmat, vmat, cmat, eidx)


def _attn_kernel(q_ref, f_ref, cq_ref, kg_ref, vg_ref, cg_ref, mask_ref,
                 wpos_ref, bpos_ref, bd_ref, rept_ref, wot_ref, g_ref, b_ref,
                 out_ref):
    q = q_ref[...]
    cq = cq_ref[...]
    wpos_b = wpos_ref[...].astype(jnp.bfloat16)
    bd = bd_ref[...]
    hi = jax.lax.Precision.HIGHEST
    ss = []
    m = jnp.full((QB, NHEAD), -jnp.inf, jnp.float32)
    for t in range(ESLOT):
        prod = q * kg_ref[t]
        s = jnp.dot(prod, bd, precision=hi) * SCALE
        rel = cq - cg_ref[t][:, :16]
        pe = jnp.dot(rel.astype(jnp.bfloat16), wpos_b,
                     preferred_element_type=jnp.float32)
        pe = pe + bpos_ref[...]
        s = s + pe + mask_ref[t]
        ss.append(s)
        m = jnp.maximum(m, s)
    acc = jnp.zeros((QB, C), jnp.float32)
    den = jnp.zeros((QB, NHEAD), jnp.float32)
    rept = rept_ref[...]
    for t in range(ESLOT):
        ex = jnp.exp(ss[t] - m)
        den = den + ex
        wex = jnp.dot(ex, rept, precision=hi)
        acc = acc + vg_ref[t] * wex
    deninv = jnp.dot(den, rept, precision=hi)
    out = acc / deninv
    of = jnp.dot(out.astype(jnp.bfloat16), wot_ref[...].astype(jnp.bfloat16),
                 preferred_element_type=jnp.float32)
    resid = of + f_ref[...]
    mu = jnp.mean(resid, axis=-1, keepdims=True)
    var = jnp.mean((resid - mu) ** 2, axis=-1, keepdims=True)
    normed = (resid - mu) / jnp.sqrt(var + 1e-5)
    out_ref[...] = normed * g_ref[...] + b_ref[...]


def _attention(q_pad, feat_pad, coords_pad, kg3, vg3, cg3, mask3,
               wpos_pad, bpos2, bd, rept, wot, gamma2, beta2):
    return pl.pallas_call(
        _attn_kernel,
        grid=(NBLK,),
        in_specs=[
            pl.BlockSpec((QB, C), lambda i: (i, 0)),
            pl.BlockSpec((QB, C), lambda i: (i, 0)),
            pl.BlockSpec((QB, 16), lambda i: (i, 0)),
            pl.BlockSpec((ESLOT, QB, C), lambda i: (0, i, 0)),
            pl.BlockSpec((ESLOT, QB, C), lambda i: (0, i, 0)),
            pl.BlockSpec((ESLOT, QB, 128), lambda i: (0, i, 0)),
            pl.BlockSpec((ESLOT, QB, NHEAD), lambda i: (0, i, 0)),
            pl.BlockSpec((16, NHEAD), lambda i: (0, 0)),
            pl.BlockSpec((1, NHEAD), lambda i: (0, 0)),
            pl.BlockSpec((C, NHEAD), lambda i: (0, 0)),
            pl.BlockSpec((NHEAD, C), lambda i: (0, 0)),
            pl.BlockSpec((C, C), lambda i: (0, 0)),
            pl.BlockSpec((1, C), lambda i: (0, 0)),
            pl.BlockSpec((1, C), lambda i: (0, 0)),
        ],
        out_specs=pl.BlockSpec((QB, C), lambda i: (i, 0)),
        out_shape=jax.ShapeDtypeStruct((NPAD, C), jnp.float32),
    )(q_pad, feat_pad, coords_pad, kg3, vg3, cg3, mask3,
      wpos_pad, bpos2, bd, rept, wot, gamma2, beta2)


def _edge_tables():
    ek = jax.random.key(12345)
    extra_q = jax.random.randint(jax.random.fold_in(ek, 0), (N * EXTRA_K,), 0, N,
                                 dtype=jnp.int32)
    extra_n = jax.random.randint(jax.random.fold_in(ek, 1), (N * EXTRA_K,), 0, N,
                                 dtype=jnp.int32)
    order = jnp.argsort(extra_q, stable=True)
    sq_ = extra_q[order]
    sn_ = extra_n[order]
    start = jnp.searchsorted(sq_, jnp.arange(N, dtype=jnp.int32), side="left")
    pos = jnp.arange(N * EXTRA_K, dtype=jnp.int32) - start[sq_]
    ext_nbr = jnp.zeros((NPAD, KNN_K), jnp.int32).at[sq_, pos].set(sn_)
    cnt = jnp.zeros((NPAD,), jnp.int32).at[sq_].add(1)
    tidx = jnp.arange(ESLOT, dtype=jnp.int32)[:, None]
    valid = (tidx < KNN_K + cnt[None, :]) & (jnp.arange(NPAD)[None, :] < N)
    mask3 = jnp.where(valid, 0.0, NEG).astype(jnp.float32)
    mask3 = jnp.broadcast_to(mask3[:, :, None], (ESLOT, NPAD, NHEAD))
    return ext_nbr, mask3


def kernel(features, coords, Wq, Wk, Wv, Wo, Wpos, bpos, gamma, beta):
    n = features.shape[0]
    pad = NPAD - n
    feat_pad = jnp.pad(features, ((0, pad), (0, 0)))
    coords_pad = jnp.pad(coords, ((0, pad), (0, 13)))
    sq = jnp.sum(coords * coords, axis=1)
    sq_col = jnp.pad(sq, (0, pad)).reshape(NPAD, 1)
    sq_row = sq.reshape(1, N)
    coordsT = coords.T

    wqkv_t = jnp.concatenate([Wq.T, Wk.T, Wv.T], axis=1)
    qkv = _project(feat_pad, wqkv_t)
    q_pad = qkv[:, :C]
    k_pad = qkv[:, C:2 * C]
    v_pad = qkv[:, 2 * C:]

    knn = _knn_idx(coords_pad, coordsT, sq_col, sq_row)

    ext_nbr, mask3 = _edge_tables()
    etab = jnp.concatenate([knn.T, ext_nbr.T], axis=0)
    eidx = etab.reshape(EROWS)
    cpad128 = jnp.pad(coords_pad, ((0, 0), (0, 112)))
    kg, vg, cg = _sc_gather(k_pad, v_pad, cpad128, eidx)
    kg3 = kg.reshape(ESLOT, NPAD, C)
    vg3 = vg.reshape(ESLOT, NPAD, C)
    cg3 = cg.reshape(ESLOT, NPAD, 128)

    wpos_pad = jnp.pad(Wpos.T, ((0, 13), (0, 0)))
    bd = (jnp.arange(C)[:, None] // DIM ==
          jnp.arange(NHEAD)[None, :]).astype(jnp.float32)
    rept = bd.T
    normed = _attention(q_pad, feat_pad, coords_pad, kg3, vg3, cg3, mask3,
                        wpos_pad, bpos.reshape(1, NHEAD), bd, rept, Wo.T,
                        gamma.reshape(1, C), beta.reshape(1, C))
    return normed[:n]
